# baseline (speedup 1.0000x reference)
import functools

import jax
import jax.numpy as jnp
from jax import lax
from jax.experimental import pallas as pl
from jax.experimental.pallas import tpu as pltpu
from jax.experimental.pallas import tpu_sc as plsc

BATCH = 16384
D = 32
NW = 32
BPW = BATCH // NW
L = 16
TILES = 7813
TPW = 245
K = 8
CAPM = 2048
CAPV = 768
CAPW = 256


def _phase1_body(items_r, users_r, items_embT_r, users_embT_r,
                 vals_a_r, vals_b_r, loc_a_r, loc_b_r,
                 idx_all, mine_r, mine_i, blks, vals_v, loc_v,
                 wl_l, wl_b, wl_i, sem):
    wid = lax.axis_index("s") * 2 + lax.axis_index("c")
    t0 = wid * TPW
    nt = jnp.minimum(TPW, TILES - t0)
    lo = t0 * 128
    hi = lo + nt * 128
    lane = lax.iota(jnp.int32, L)
    zeros_i = jnp.zeros((L,), jnp.int32)

    for idx_r, tbl_r, vals_out_r, loc_out_r in (
            (items_r, items_embT_r, vals_a_r, loc_a_r),
            (users_r, users_embT_r, vals_b_r, loc_b_r)):
        pltpu.sync_copy(idx_r, idx_all)

        def zero_loc(k, carry):
            loc_v[pl.ds(k * L, L)] = zeros_i
            return carry

        lax.fori_loop(0, BATCH // L, zero_loc, 0)

        def comp(k, n):
            for h in range(2):
                kk = k * 2 + h
                rv = idx_all[pl.ds(kk * L, L)]
                m = (rv >= lo) & (rv < hi)
                mi = m.astype(jnp.int32)
                pos = n + plsc.cumsum(mi) - mi
                m = m & (pos < CAPM)
                plsc.store_scatter(mine_r, [pos], rv, mask=m)
                plsc.store_scatter(mine_i, [pos], kk * L + lane, mask=m)
                n = n + plsc.all_reduce_population_count(m)[0]
            return n

        n = lax.fori_loop(0, BATCH // L // 2, comp, 0)
        nk = (n + L - 1) // L
        nwv = (nt + K - 1) // K

        def fire(wv, buf):
            for kk in range(K):
                t = jnp.minimum(t0 + wv * K + kk, TILES - 1)
                start = pl.multiple_of(t * 128, 128)
                pltpu.async_copy(
                    tbl_r.at[pl.ds(0, D), pl.ds(start, 128)],
                    blks.at[buf, kk], sem)

        fire(0, 0)

        def wave_iter(wv, slot_count):
            par = wv & 1
            for kk in range(K):
                pltpu.make_async_copy(
                    tbl_r.at[pl.ds(0, D), pl.ds(0, 128)], blks.at[par, kk],
                    sem).wait()

            @pl.when(wv + 1 < nwv)
            def _():
                fire(wv + 1, 1 - par)

            tbase = t0 + wv * K
            parv = jnp.broadcast_to(par, (L,))

            def scan_k(k, wn):
                for h in range(2):
                    kk = k * 2 + h
                    rv = mine_r[pl.ds(kk * L, L)]
                    iv = mine_i[pl.ds(kk * L, L)]
                    bs = (rv >> 7) - tbase
                    m = (bs >= 0) & (bs < K) & ((kk * L + lane) < n)
                    c = plsc.all_reduce_population_count(m)[0]

                    @pl.when(c > 0)
                    def _():
                        mi = m.astype(jnp.int32)
                        pos = wn + plsc.cumsum(mi) - mi
                        mm = m & (pos < CAPW)
                        plsc.store_scatter(wl_l, [pos], rv & 127, mask=mm)
                        plsc.store_scatter(wl_b, [pos], bs, mask=mm)
                        plsc.store_scatter(wl_i, [pos], iv, mask=mm)

                    wn = wn + c
                return wn

            wn = lax.fori_loop(0, (nk + 1) // 2, scan_k, 0)

            def extract_k(k, carry):
                valid = (k * L + lane) < wn
                lv = jnp.where(valid, wl_l[pl.ds(k * L, L)], 0)
                bsv = jnp.where(valid, wl_b[pl.ds(k * L, L)], 0)
                iv = jnp.where(valid, wl_i[pl.ds(k * L, L)], 0)
                slots = slot_count + k * L + lane
                mm = valid & (slots < CAPV)
                base32 = slots * D
                for d in range(D):
                    vals_d = plsc.load_gather(
                        blks, [parv, bsv, jnp.broadcast_to(d, (L,)), lv])
                    plsc.store_scatter(vals_v, [base32 + d], vals_d,
                                       mask=mm)
                plsc.store_scatter(loc_v, [iv],
                                   1 + wid * CAPV + slots, mask=mm)
                return carry

            lax.fori_loop(0, (wn + L - 1) // L, extract_k, 0)
            return slot_count + wn

        lax.fori_loop(0, nwv, wave_iter, 0)

        pltpu.sync_copy(vals_v, vals_out_r.at[pl.ds(wid * CAPV * D, CAPV * D)])
        pltpu.sync_copy(loc_v, loc_out_r.at[pl.ds(wid * BATCH, BATCH)])


def _phase2_body(vals_a_r, vals_b_r, loc_a_r, loc_b_r, out_r,
                 loc_tmps, loc_acc_a, loc_acc_b, rows_a, rows_b, out_v, sem):
    wid = lax.axis_index("s") * 2 + lax.axis_index("c")
    base = wid * BPW
    lane = lax.iota(jnp.int32, L)

    for loc_r, loc_acc in ((loc_a_r, loc_acc_a), (loc_b_r, loc_acc_b)):
        cps = []
        for v in range(NW):
            cps.append(pltpu.async_copy(
                loc_r.at[pl.ds(v * BATCH + base, BPW)],
                loc_tmps.at[v], sem))
        for cp in cps:
            cp.wait()

        def acc_k(k, carry):
            s = jnp.zeros((L,), jnp.int32)
            for v in range(NW):
                s = s + loc_tmps[v, pl.ds(k * L, L)]
            loc_acc[pl.ds(k * L, L)] = s
            return carry

        lax.fori_loop(0, BPW // L, acc_k, 0)

    def fetch(g, carry):
        ea = jnp.maximum(loc_acc_a[pl.ds(g * L, L)] - 1, 0)
        eb = jnp.maximum(loc_acc_b[pl.ds(g * L, L)] - 1, 0)
        for j in range(L):
            c = g * L + j
            pltpu.async_copy(vals_a_r.at[pl.ds(ea[j] * D, D)],
                             rows_a.at[pl.ds(c * D, D)], sem)
            pltpu.async_copy(vals_b_r.at[pl.ds(eb[j] * D, D)],
                             rows_b.at[pl.ds(c * D, D)], sem)
        return carry

    lax.fori_loop(0, BPW // L, fetch, 0)
    pltpu.make_async_copy(vals_a_r.at[pl.ds(0, BPW * D)], rows_a, sem).wait()
    pltpu.make_async_copy(vals_b_r.at[pl.ds(0, BPW * D)], rows_b, sem).wait()

    lane32 = lane * D

    def group(g, carry):
        ridx = g * (L * D) + lane32
        acc = jnp.zeros((L,), jnp.float32)
        for d in range(D):
            pa = plsc.load_gather(rows_a, [ridx + d])
            pb = plsc.load_gather(rows_b, [ridx + d])
            acc = acc + pa * pb
        sig = 1.0 / (1.0 + jnp.exp(-acc))
        out_v[pl.ds(g * L, L)] = sig
        return carry

    lax.fori_loop(0, BPW // L, group, 0)

    pltpu.sync_copy(out_v, out_r.at[pl.ds(base, BPW)])


@jax.jit
def _gmf(items, users, items_embedding, users_embedding):
    mesh = plsc.VectorSubcoreMesh(core_axis_name="c", subcore_axis_name="s")
    cparams = pltpu.CompilerParams(needs_layout_passes=False)

    phase1 = functools.partial(
        pl.kernel,
        mesh=mesh,
        out_type=[
            jax.ShapeDtypeStruct((NW * CAPV * D,), jnp.float32),
            jax.ShapeDtypeStruct((NW * CAPV * D,), jnp.float32),
            jax.ShapeDtypeStruct((NW * BATCH,), jnp.int32),
            jax.ShapeDtypeStruct((NW * BATCH,), jnp.int32),
        ],
        scratch_types=[
            pltpu.VMEM((BATCH,), jnp.int32),
            pltpu.VMEM((CAPM,), jnp.int32),
            pltpu.VMEM((CAPM,), jnp.int32),
            pltpu.VMEM((2, K, D, 128), jnp.float32),
            pltpu.VMEM((CAPV * D,), jnp.float32),
            pltpu.VMEM((BATCH,), jnp.int32),
            pltpu.VMEM((CAPW,), jnp.int32),
            pltpu.VMEM((CAPW,), jnp.int32),
            pltpu.VMEM((CAPW,), jnp.int32),
            pltpu.SemaphoreType.DMA,
        ],
        compiler_params=cparams,
    )(_phase1_body)

    phase2 = functools.partial(
        pl.kernel,
        mesh=mesh,
        out_type=jax.ShapeDtypeStruct((BATCH,), jnp.float32),
        scratch_types=[
            pltpu.VMEM((NW, BPW), jnp.int32),
            pltpu.VMEM((BPW,), jnp.int32),
            pltpu.VMEM((BPW,), jnp.int32),
            pltpu.VMEM((BPW * D,), jnp.float32),
            pltpu.VMEM((BPW * D,), jnp.float32),
            pltpu.VMEM((BPW,), jnp.float32),
            pltpu.SemaphoreType.DMA,
        ],
        compiler_params=cparams,
    )(_phase2_body)

    va, vb, la, lb = phase1(items, users,
                            items_embedding.T, users_embedding.T)
    return phase2(va, vb, la, lb)


def kernel(items, users, items_embedding, users_embedding):
    return _gmf(items.astype(jnp.int32), users.astype(jnp.int32),
                items_embedding, users_embedding)


# speedup vs baseline: 1.0090x; 1.0090x over previous
import functools

import jax
import jax.numpy as jnp
from jax import lax
from jax.experimental import pallas as pl
from jax.experimental.pallas import tpu as pltpu
from jax.experimental.pallas import tpu_sc as plsc

BATCH = 16384
D = 32
NW = 32
BPW = BATCH // NW
L = 16
TILES = 7813
TPW = 245
K = 8
CAPM = 2048
CAPV = 768
CAPW = 256


def _phase1_body(items_r, users_r, items_embT_r, users_embT_r,
                 vals_a_r, vals_b_r, loc_a_r, loc_b_r,
                 idx_all, mine_r, mine_i, blks, vals_v, loc_v,
                 wl_l, wl_i, sem):
    wid = lax.axis_index("s") * 2 + lax.axis_index("c")
    t0 = wid * TPW
    nt = jnp.minimum(TPW, TILES - t0)
    lo = t0 * 128
    hi = lo + nt * 128
    lane = lax.iota(jnp.int32, L)
    zeros_i = jnp.zeros((L,), jnp.int32)

    for idx_r, tbl_r, vals_out_r, loc_out_r in (
            (items_r, items_embT_r, vals_a_r, loc_a_r),
            (users_r, users_embT_r, vals_b_r, loc_b_r)):
        pltpu.sync_copy(idx_r, idx_all)

        def zero_loc(k, carry):
            loc_v[pl.ds(k * L, L)] = zeros_i
            return carry

        lax.fori_loop(0, BATCH // L, zero_loc, 0)

        def comp(k, n):
            for h in range(2):
                kk = k * 2 + h
                rv = idx_all[pl.ds(kk * L, L)]
                m = (rv >= lo) & (rv < hi)
                mi = m.astype(jnp.int32)
                pos = n + plsc.cumsum(mi) - mi
                m = m & (pos < CAPM)
                plsc.store_scatter(mine_r, [pos], rv, mask=m)
                plsc.store_scatter(mine_i, [pos], kk * L + lane, mask=m)
                n = n + plsc.all_reduce_population_count(m)[0]
            return n

        n = lax.fori_loop(0, BATCH // L // 2, comp, 0)
        nk = (n + L - 1) // L
        nwv = (nt + K - 1) // K
        KW = K * 128

        def wstart_of(wv):
            return pl.multiple_of(
                jnp.minimum((t0 + wv * K) * 128, TILES * 128 - KW), 128)

        def fire(wv, buf):
            start = wstart_of(wv)
            for g in range(D // 8):
                pltpu.async_copy(
                    tbl_r.at[pl.ds(g * 8, 8), pl.ds(start, KW)],
                    blks.at[buf, g], sem)

        fire(0, 0)

        def wave_iter(wv, slot_count):
            par = wv & 1
            for g in range(D // 8):
                pltpu.make_async_copy(
                    tbl_r.at[pl.ds(0, 8), pl.ds(0, KW)], blks.at[par, g],
                    sem).wait()

            @pl.when(wv + 1 < nwv)
            def _():
                fire(wv + 1, 1 - par)

            tbase = t0 + wv * K
            wstart = wstart_of(wv)
            parv = jnp.broadcast_to(par, (L,))

            def scan_k(k, wn):
                for h in range(2):
                    kk = k * 2 + h
                    rv = mine_r[pl.ds(kk * L, L)]
                    iv = mine_i[pl.ds(kk * L, L)]
                    bs = (rv >> 7) - tbase
                    m = (bs >= 0) & (bs < K) & ((kk * L + lane) < n)
                    c = plsc.all_reduce_population_count(m)[0]

                    @pl.when(c > 0)
                    def _():
                        mi = m.astype(jnp.int32)
                        pos = wn + plsc.cumsum(mi) - mi
                        mm = m & (pos < CAPW)
                        plsc.store_scatter(wl_l, [pos], rv - wstart, mask=mm)
                        plsc.store_scatter(wl_i, [pos], iv, mask=mm)

                    wn = wn + c
                return wn

            wn = lax.fori_loop(0, (nk + 1) // 2, scan_k, 0)

            def extract_k(k, carry):
                valid = (k * L + lane) < wn
                cv = jnp.where(valid, wl_l[pl.ds(k * L, L)], 0)
                iv = jnp.where(valid, wl_i[pl.ds(k * L, L)], 0)
                slots = slot_count + k * L + lane
                mm = valid & (slots < CAPV)
                base32 = slots * D
                for d in range(D):
                    vals_d = plsc.load_gather(
                        blks, [parv, jnp.broadcast_to(d // 8, (L,)),
                               jnp.broadcast_to(d % 8, (L,)), cv])
                    plsc.store_scatter(vals_v, [base32 + d], vals_d,
                                       mask=mm)
                plsc.store_scatter(loc_v, [iv],
                                   1 + wid * CAPV + slots, mask=mm)
                return carry

            lax.fori_loop(0, (wn + L - 1) // L, extract_k, 0)
            return slot_count + wn

        lax.fori_loop(0, nwv, wave_iter, 0)

        pltpu.sync_copy(vals_v, vals_out_r.at[pl.ds(wid * CAPV * D, CAPV * D)])
        pltpu.sync_copy(loc_v, loc_out_r.at[pl.ds(wid * BATCH, BATCH)])


def _phase2_body(vals_a_r, vals_b_r, loc_a_r, loc_b_r, out_r,
                 loc_tmps, loc_acc_a, loc_acc_b, rows_a, rows_b, out_v, sem):
    wid = lax.axis_index("s") * 2 + lax.axis_index("c")
    base = wid * BPW
    lane = lax.iota(jnp.int32, L)

    for loc_r, loc_acc in ((loc_a_r, loc_acc_a), (loc_b_r, loc_acc_b)):
        cps = []
        for v in range(NW):
            cps.append(pltpu.async_copy(
                loc_r.at[pl.ds(v * BATCH + base, BPW)],
                loc_tmps.at[v], sem))
        for cp in cps:
            cp.wait()

        def acc_k(k, carry):
            s = jnp.zeros((L,), jnp.int32)
            for v in range(NW):
                s = s + loc_tmps[v, pl.ds(k * L, L)]
            loc_acc[pl.ds(k * L, L)] = s
            return carry

        lax.fori_loop(0, BPW // L, acc_k, 0)

    def fetch(g, carry):
        ea = jnp.maximum(loc_acc_a[pl.ds(g * L, L)] - 1, 0)
        eb = jnp.maximum(loc_acc_b[pl.ds(g * L, L)] - 1, 0)
        for j in range(L):
            c = g * L + j
            pltpu.async_copy(vals_a_r.at[pl.ds(ea[j] * D, D)],
                             rows_a.at[pl.ds(c * D, D)], sem)
            pltpu.async_copy(vals_b_r.at[pl.ds(eb[j] * D, D)],
                             rows_b.at[pl.ds(c * D, D)], sem)
        return carry

    lax.fori_loop(0, BPW // L, fetch, 0)
    pltpu.make_async_copy(vals_a_r.at[pl.ds(0, BPW * D)], rows_a, sem).wait()
    pltpu.make_async_copy(vals_b_r.at[pl.ds(0, BPW * D)], rows_b, sem).wait()

    lane32 = lane * D

    def group(g, carry):
        ridx = g * (L * D) + lane32
        acc = jnp.zeros((L,), jnp.float32)
        for d in range(D):
            pa = plsc.load_gather(rows_a, [ridx + d])
            pb = plsc.load_gather(rows_b, [ridx + d])
            acc = acc + pa * pb
        sig = 1.0 / (1.0 + jnp.exp(-acc))
        out_v[pl.ds(g * L, L)] = sig
        return carry

    lax.fori_loop(0, BPW // L, group, 0)

    pltpu.sync_copy(out_v, out_r.at[pl.ds(base, BPW)])


@jax.jit
def _gmf(items, users, items_embedding, users_embedding):
    mesh = plsc.VectorSubcoreMesh(core_axis_name="c", subcore_axis_name="s")
    cparams = pltpu.CompilerParams(needs_layout_passes=False)

    phase1 = functools.partial(
        pl.kernel,
        mesh=mesh,
        out_type=[
            jax.ShapeDtypeStruct((NW * CAPV * D,), jnp.float32),
            jax.ShapeDtypeStruct((NW * CAPV * D,), jnp.float32),
            jax.ShapeDtypeStruct((NW * BATCH,), jnp.int32),
            jax.ShapeDtypeStruct((NW * BATCH,), jnp.int32),
        ],
        scratch_types=[
            pltpu.VMEM((BATCH,), jnp.int32),
            pltpu.VMEM((CAPM,), jnp.int32),
            pltpu.VMEM((CAPM,), jnp.int32),
            pltpu.VMEM((2, D // 8, 8, K * 128), jnp.float32),
            pltpu.VMEM((CAPV * D,), jnp.float32),
            pltpu.VMEM((BATCH,), jnp.int32),
            pltpu.VMEM((CAPW,), jnp.int32),
            pltpu.VMEM((CAPW,), jnp.int32),
            pltpu.SemaphoreType.DMA,
        ],
        compiler_params=cparams,
    )(_phase1_body)

    phase2 = functools.partial(
        pl.kernel,
        mesh=mesh,
        out_type=jax.ShapeDtypeStruct((BATCH,), jnp.float32),
        scratch_types=[
            pltpu.VMEM((NW, BPW), jnp.int32),
            pltpu.VMEM((BPW,), jnp.int32),
            pltpu.VMEM((BPW,), jnp.int32),
            pltpu.VMEM((BPW * D,), jnp.float32),
            pltpu.VMEM((BPW * D,), jnp.float32),
            pltpu.VMEM((BPW,), jnp.float32),
            pltpu.SemaphoreType.DMA,
        ],
        compiler_params=cparams,
    )(_phase2_body)

    va, vb, la, lb = phase1(items, users,
                            items_embedding.T, users_embedding.T)
    return phase2(va, vb, la, lb)


def kernel(items, users, items_embedding, users_embedding):
    return _gmf(items.astype(jnp.int32), users.astype(jnp.int32),
                items_embedding, users_embedding)
